# Initial kernel scaffold; baseline (speedup 1.0000x reference)
import jax
import jax.numpy as jnp
from jax.experimental import pallas as pl


def kernel(x, edge_index, edge_attr, W_lin0, b_lin0, W_lin1, b_lin1, g0_W1, g0_b1, g0_W2, g0_b2, g1_W1, g1_b1, g1_W2, g1_b2):
    raise NotImplementedError("write your pallas kernel here")


# speedup vs baseline: 2.6645x; 2.6645x over previous
import functools

import jax
import jax.numpy as jnp
from jax import lax
from jax.experimental import pallas as pl
from jax.experimental.pallas import tpu as pltpu
from jax.experimental.pallas import tpu_sc as plsc

N, E, D = 10000, 320000, 128
NC, NS = 2, 16
NW = NC * NS
EPW = E // NW
CH = 80
NCHUNK = EPW // CH
NPAD = 10112
RPT = NPAD // NS
LANES = 16


def _edge_mlp_body(a_ref, w0_ref, b0_ref, w1_ref, b1_ref, e_ref):
    a = a_ref[...]
    t = jnp.maximum(a * w0_ref[...] + b0_ref[...], 0.0)
    e_ref[...] = (
        jnp.dot(t, w1_ref[...], preferred_element_type=jnp.float32)
        + b1_ref[...]
    )


def _edge_mlp(edge_attr, w0, b0, w1, b1):
    BE = 4000
    grid = E // BE
    return pl.pallas_call(
        _edge_mlp_body,
        grid=(grid,),
        in_specs=[
            pl.BlockSpec((BE, 1), lambda i: (i, 0)),
            pl.BlockSpec((1, D), lambda i: (0, 0)),
            pl.BlockSpec((1, D), lambda i: (0, 0)),
            pl.BlockSpec((D, D), lambda i: (0, 0)),
            pl.BlockSpec((1, D), lambda i: (0, 0)),
        ],
        out_specs=pl.BlockSpec((BE, D), lambda i: (i, 0)),
        out_shape=jax.ShapeDtypeStruct((E, D), jnp.float32),
    )(edge_attr, w0, b0.reshape(1, D), w1, b1.reshape(1, D))


def _node_mlp_body(x_ref, p_ref, w1_ref, b1_ref, w2_ref, b2_ref, o_ref):
    h = x_ref[...] + p_ref[0] + p_ref[1]
    h = jnp.maximum(
        jnp.dot(h, w1_ref[...], preferred_element_type=jnp.float32)
        + b1_ref[...], 0.0)
    o_ref[...] = jnp.maximum(
        jnp.dot(h, w2_ref[...], preferred_element_type=jnp.float32)
        + b2_ref[...], 0.0)


def _node_mlp(x, p, w1, b1, w2, b2):
    BN = 2000
    grid = N // BN
    return pl.pallas_call(
        _node_mlp_body,
        grid=(grid,),
        in_specs=[
            pl.BlockSpec((BN, D), lambda i: (i, 0)),
            pl.BlockSpec((NC, BN, D), lambda i: (0, i, 0)),
            pl.BlockSpec((D, D), lambda i: (0, 0)),
            pl.BlockSpec((1, D), lambda i: (0, 0)),
            pl.BlockSpec((D, D), lambda i: (0, 0)),
            pl.BlockSpec((1, D), lambda i: (0, 0)),
        ],
        out_specs=pl.BlockSpec((BN, D), lambda i: (i, 0)),
        out_shape=jax.ShapeDtypeStruct((N, D), jnp.float32),
    )(x, p, w1, b1.reshape(1, D), w2, b2.reshape(1, D))


def _sc_aggr_body(x_hbm, src_hbm, dst_hbm, e_hbm, zeros_hbm, out_hbm,
                  src_v, dst_v, e_v, xg_v, aggr_sh, sem):
    c = lax.axis_index("c")
    s = lax.axis_index("s")
    wid = s * NC + c

    pltpu.sync_copy(zeros_hbm.at[pl.ds(s * RPT, RPT)],
                    aggr_sh.at[pl.ds(s * RPT, RPT)])
    plsc.subcore_barrier()

    def chunk(j, carry):
        base = wid * EPW + j * CH
        pltpu.sync_copy(src_hbm.at[pl.ds(base, CH)], src_v)
        pltpu.sync_copy(dst_hbm.at[pl.ds(base, CH)], dst_v)
        pltpu.async_copy(x_hbm.at[src_v], xg_v, sem).wait()
        pltpu.sync_copy(e_hbm.at[pl.ds(base, CH)], e_v)

        def row(r, carry2):
            for k in range(D // LANES):
                sl = pl.ds(k * LANES, LANES)
                xg_v[r, sl] = jnp.maximum(xg_v[r, sl] + e_v[r, sl], 0.0)
            return carry2

        lax.fori_loop(0, CH, row, 0)
        pltpu.sync_copy(xg_v, aggr_sh.at[dst_v], add=True)
        return carry

    lax.fori_loop(0, NCHUNK, chunk, 0)
    plsc.subcore_barrier()
    pltpu.sync_copy(aggr_sh.at[pl.ds(s * RPT, RPT)],
                    out_hbm.at[c, pl.ds(s * RPT, RPT)])


_sc_aggr = pl.kernel(
    _sc_aggr_body,
    out_type=jax.ShapeDtypeStruct((NC, NPAD, D), jnp.float32),
    mesh=plsc.VectorSubcoreMesh(
        core_axis_name="c", subcore_axis_name="s",
        num_cores=NC, num_subcores=NS),
    scratch_types=[
        pltpu.VMEM((CH,), jnp.int32),
        pltpu.VMEM((CH,), jnp.int32),
        pltpu.VMEM((CH, D), jnp.float32),
        pltpu.VMEM((CH, D), jnp.float32),
        pltpu.VMEM_SHARED((NPAD, D), jnp.float32),
        pltpu.SemaphoreType.DMA,
    ],
)


def kernel(x, edge_index, edge_attr, W_lin0, b_lin0, W_lin1, b_lin1,
           g0_W1, g0_b1, g0_W2, g0_b2, g1_W1, g1_b1, g1_W2, g1_b2):
    src = edge_index[0].astype(jnp.int32)
    dst = edge_index[1].astype(jnp.int32)
    zeros = jnp.zeros((NPAD, D), jnp.float32)

    e = _edge_mlp(edge_attr, W_lin0, b_lin0, W_lin1, b_lin1)

    p = _sc_aggr(x, src, dst, e, zeros)
    h = _node_mlp(x, p, g0_W1, g0_b1, g0_W2, g0_b2)

    p = _sc_aggr(h, src, dst, e, zeros)
    h = _node_mlp(h, p, g1_W1, g1_b1, g1_W2, g1_b2)
    return h


# speedup vs baseline: 5.0438x; 1.8930x over previous
import functools

import jax
import jax.numpy as jnp
from jax import lax
from jax.experimental import pallas as pl
from jax.experimental.pallas import tpu as pltpu
from jax.experimental.pallas import tpu_sc as plsc

N, E, D = 10000, 320000, 128
NC, NS = 2, 16
NW = NC * NS
EPW = E // NW
CH = 40
NCHUNK = EPW // CH
NPAD = 10112
RPT = NPAD // NS
LANES = 16


def _edge_mlp_body(a_ref, w0_ref, b0_ref, w1_ref, b1_ref, e_ref):
    a = a_ref[...]
    t = jnp.maximum(a * w0_ref[...] + b0_ref[...], 0.0)
    e_ref[...] = (
        jnp.dot(t, w1_ref[...], preferred_element_type=jnp.float32)
        + b1_ref[...]
    )


def _edge_mlp(edge_attr, w0, b0, w1, b1):
    BE = 4000
    grid = E // BE
    return pl.pallas_call(
        _edge_mlp_body,
        grid=(grid,),
        in_specs=[
            pl.BlockSpec((BE, 1), lambda i: (i, 0)),
            pl.BlockSpec((1, D), lambda i: (0, 0)),
            pl.BlockSpec((1, D), lambda i: (0, 0)),
            pl.BlockSpec((D, D), lambda i: (0, 0)),
            pl.BlockSpec((1, D), lambda i: (0, 0)),
        ],
        out_specs=pl.BlockSpec((BE, D), lambda i: (i, 0)),
        out_shape=jax.ShapeDtypeStruct((E, D), jnp.float32),
    )(edge_attr, w0, b0.reshape(1, D), w1, b1.reshape(1, D))


def _node_mlp_body(x_ref, p_ref, w1_ref, b1_ref, w2_ref, b2_ref, o_ref):
    h = x_ref[...] + p_ref[0] + p_ref[1]
    h = jnp.maximum(
        jnp.dot(h, w1_ref[...], preferred_element_type=jnp.float32)
        + b1_ref[...], 0.0)
    o_ref[...] = jnp.maximum(
        jnp.dot(h, w2_ref[...], preferred_element_type=jnp.float32)
        + b2_ref[...], 0.0)


def _node_mlp(x, p, w1, b1, w2, b2):
    BN = 2000
    grid = N // BN
    return pl.pallas_call(
        _node_mlp_body,
        grid=(grid,),
        in_specs=[
            pl.BlockSpec((BN, D), lambda i: (i, 0)),
            pl.BlockSpec((NC, BN, D), lambda i: (0, i, 0)),
            pl.BlockSpec((D, D), lambda i: (0, 0)),
            pl.BlockSpec((1, D), lambda i: (0, 0)),
            pl.BlockSpec((D, D), lambda i: (0, 0)),
            pl.BlockSpec((1, D), lambda i: (0, 0)),
        ],
        out_specs=pl.BlockSpec((BN, D), lambda i: (i, 0)),
        out_shape=jax.ShapeDtypeStruct((N, D), jnp.float32),
    )(x, p, w1, b1.reshape(1, D), w2, b2.reshape(1, D))


def _sc_aggr_body(x_hbm, src_hbm, dst_hbm, e_hbm, zeros_hbm, out_hbm,
                  srcall_v, dstA, dstB, eA, eB, xgA, xgB, aggr_sh,
                  sga, sea, sgb, seb, sidx):
    c = lax.axis_index("c")
    s = lax.axis_index("s")
    wid = s * NC + c

    pltpu.async_copy(src_hbm.at[pl.ds(wid * EPW, EPW)], srcall_v, sidx)
    pltpu.sync_copy(zeros_hbm.at[pl.ds(s * RPT, RPT)],
                    aggr_sh.at[pl.ds(s * RPT, RPT)])
    pltpu.make_async_copy(src_hbm.at[pl.ds(0, EPW)], srcall_v, sidx).wait()
    plsc.subcore_barrier()

    def fire(j, xg_v, e_v, dst_v, sg, se):
        base = wid * EPW + j * CH
        pltpu.async_copy(x_hbm.at[srcall_v.at[pl.ds(j * CH, CH)]], xg_v, sg)
        pltpu.async_copy(e_hbm.at[pl.ds(base, CH)], e_v, se)
        pltpu.async_copy(dst_hbm.at[pl.ds(base, CH)], dst_v, se)

    def process(j, xg_v, e_v, dst_v, sg, se):
        pltpu.make_async_copy(x_hbm.at[pl.ds(0, CH)], xg_v, sg).wait()
        pltpu.make_async_copy(e_hbm.at[pl.ds(0, CH)], e_v, se).wait()
        pltpu.make_async_copy(dst_hbm.at[pl.ds(0, CH)], dst_v, se).wait()

        @plsc.parallel_loop(0, CH, step=1, unroll=4)
        def row(r):
            for k in range(D // LANES):
                sl = pl.ds(k * LANES, LANES)
                xg_v[r, sl] = jnp.maximum(xg_v[r, sl] + e_v[r, sl], 0.0)

        pltpu.sync_copy(xg_v, aggr_sh.at[dst_v], add=True)

    fire(0, xgA, eA, dstA, sga, sea)
    fire(1, xgB, eB, dstB, sgb, seb)

    def outer(jj, carry):
        j = 2 * jj
        process(j, xgA, eA, dstA, sga, sea)
        fire(j + 2, xgA, eA, dstA, sga, sea)
        process(j + 1, xgB, eB, dstB, sgb, seb)
        fire(j + 3, xgB, eB, dstB, sgb, seb)
        return carry

    lax.fori_loop(0, NCHUNK // 2 - 1, outer, 0)
    process(NCHUNK - 2, xgA, eA, dstA, sga, sea)
    process(NCHUNK - 1, xgB, eB, dstB, sgb, seb)

    plsc.subcore_barrier()
    pltpu.sync_copy(aggr_sh.at[pl.ds(s * RPT, RPT)],
                    out_hbm.at[c, pl.ds(s * RPT, RPT)])


_sc_aggr = pl.kernel(
    _sc_aggr_body,
    out_type=jax.ShapeDtypeStruct((NC, NPAD, D), jnp.float32),
    mesh=plsc.VectorSubcoreMesh(
        core_axis_name="c", subcore_axis_name="s",
        num_cores=NC, num_subcores=NS),
    scratch_types=[
        pltpu.VMEM((EPW,), jnp.int32),
        pltpu.VMEM((CH,), jnp.int32),
        pltpu.VMEM((CH,), jnp.int32),
        pltpu.VMEM((CH, D), jnp.float32),
        pltpu.VMEM((CH, D), jnp.float32),
        pltpu.VMEM((CH, D), jnp.float32),
        pltpu.VMEM((CH, D), jnp.float32),
        pltpu.VMEM_SHARED((NPAD, D), jnp.float32),
        pltpu.SemaphoreType.DMA,
        pltpu.SemaphoreType.DMA,
        pltpu.SemaphoreType.DMA,
        pltpu.SemaphoreType.DMA,
        pltpu.SemaphoreType.DMA,
    ],
)


def kernel(x, edge_index, edge_attr, W_lin0, b_lin0, W_lin1, b_lin1,
           g0_W1, g0_b1, g0_W2, g0_b2, g1_W1, g1_b1, g1_W2, g1_b2):
    src = edge_index[0].astype(jnp.int32)
    dst = edge_index[1].astype(jnp.int32)
    zeros = jnp.zeros((NPAD, D), jnp.float32)

    e = _edge_mlp(edge_attr, W_lin0, b_lin0, W_lin1, b_lin1)

    p = _sc_aggr(x, src, dst, e, zeros)
    h = _node_mlp(x, p, g0_W1, g0_b1, g0_W2, g0_b2)

    p = _sc_aggr(h, src, dst, e, zeros)
    h = _node_mlp(h, p, g1_W1, g1_b1, g1_W2, g1_b2)
    return h
